# Initial kernel scaffold; baseline (speedup 1.0000x reference)
import jax
import jax.numpy as jnp
from jax.experimental import pallas as pl


def kernel(x, edge_index, W1, al1, ar1, b1, W2, al2, ar2, b2, l1W, l1b, l2W, l2b, l3W, l3b):
    raise NotImplementedError("write your pallas kernel here")


# speedup vs baseline: 26.3200x; 26.3200x over previous
import numpy as np

import jax
import jax.numpy as jnp
from jax.experimental import pallas as pl
from jax.experimental.pallas import tpu as pltpu

_N = 512
_E = 16384
_NH = 4
_HID = 128
_F32 = jnp.float32


def _count_body(src_ref, dst_ref, c_ref):
    ec = src_ref.shape[-1]
    n = c_ref.shape[0]
    src = src_ref[...].reshape(1, ec)
    dst = dst_ref[...].reshape(1, ec)
    iota = jax.lax.broadcasted_iota(jnp.int32, (n, ec), 0)
    src_oh = (iota == src).astype(jnp.bfloat16)
    dst_oh = (iota == dst).astype(jnp.bfloat16)
    part = jax.lax.dot_general(dst_oh, src_oh, (((1,), (1,)), ((), ())),
                               preferred_element_type=_F32)

    @pl.when(pl.program_id(0) == 0)
    def _():
        c_ref[...] = part

    @pl.when(pl.program_id(0) != 0)
    def _():
        c_ref[...] += part


def _gat_body(x_ref, w1_ref, albd1_ref, arbd1_ref, b1_ref,
              w2_ref, albd2_ref, arbd2_ref, b2_ref,
              c_ref, l1wa_ref, l1wb_ref, l1b_ref,
              a_out_ref, b_out_ref):
    n = x_ref.shape[0]
    cm = c_ref[...]
    mask = cm > 0.0

    def attention(h, albd, arbd):
        el = jnp.dot(h, albd, preferred_element_type=_F32)
        er = jnp.dot(h, arbd, preferred_element_type=_F32)
        elt = el.T
        outs = []
        for hd in range(_NH):
            hh = h[:, hd * _HID:(hd + 1) * _HID]
            e = elt[hd:hd + 1, :] + er[:, hd:hd + 1]
            e = jnp.where(e >= 0.0, e, 0.2 * e)
            emax = jnp.max(jnp.where(mask, e, -1e30), axis=1, keepdims=True)
            emax = jnp.where(emax > -1e29, emax, 0.0)
            p = jnp.where(mask, jnp.exp(e - emax), 0.0) * cm
            denom = jnp.sum(p, axis=1, keepdims=True)
            denom = jnp.where(denom > 0.0, denom, 1.0)
            alpha = p / denom
            outs.append(jnp.dot(alpha, hh, preferred_element_type=_F32))
        return outs

    h1 = jnp.dot(x_ref[...], w1_ref[...], preferred_element_type=_F32)
    o1 = attention(h1, albd1_ref[...], arbd1_ref[...])
    acts = []
    for hd in range(_NH):
        v = o1[hd] + b1_ref[...][:, hd * _HID:(hd + 1) * _HID]
        acts.append(jnp.where(v > 0.0, v, jnp.exp(jnp.minimum(v, 0.0)) - 1.0))
    h2in = jnp.concatenate(acts, axis=1)

    h2 = jnp.dot(h2in, w2_ref[...], preferred_element_type=_F32)
    o2 = attention(h2, albd2_ref[...], arbd2_ref[...])
    hm = jnp.zeros((n, _HID), _F32)
    for hd in range(_NH):
        hm = hm + (o2[hd] + h2in[:, hd * _HID:(hd + 1) * _HID]
                   + b2_ref[...][:, hd * _HID:(hd + 1) * _HID])
    hm = hm * (1.0 / _NH)

    a_out_ref[...] = (jnp.dot(hm, l1wa_ref[...], preferred_element_type=_F32)
                      + l1b_ref[...])
    b_out_ref[...] = jnp.dot(hm, l1wb_ref[...], preferred_element_type=_F32)


def _pair_body(a_ref, b_ref, w2_ref, b2_ref, w3_ref, b3_ref, out_ref):
    bi = a_ref.shape[0]
    n = b_ref.shape[0]
    z = jnp.maximum(a_ref[...][:, None, :] + b_ref[...][None, :, :], 0.0)
    z = z.reshape(bi * n, _HID)
    q = jnp.dot(z, w2_ref[...], preferred_element_type=_F32) + b2_ref[...]
    q = jnp.maximum(q, 0.0)
    s = jnp.sum(q * w3_ref[...], axis=1, keepdims=True) + b3_ref[...]
    out_ref[...] = 1.0 / (1.0 + jnp.exp(-s))


_KRON = np.kron(np.eye(_NH, dtype=np.float32), np.ones((_HID, 1), np.float32))


def kernel(x, edge_index, W1, al1, ar1, b1, W2, al2, ar2, b2,
           l1W, l1b, l2W, l2b, l3W, l3b):
    n, e = _N, _E

    nch = 8
    src3 = edge_index[0].reshape(nch, 1, e // nch)
    dst3 = edge_index[1].reshape(nch, 1, e // nch)
    C = pl.pallas_call(
        _count_body,
        grid=(nch,),
        in_specs=[pl.BlockSpec((1, 1, e // nch), lambda i: (i, 0, 0)),
                  pl.BlockSpec((1, 1, e // nch), lambda i: (i, 0, 0))],
        out_specs=pl.BlockSpec((n, n), lambda i: (0, 0)),
        out_shape=jax.ShapeDtypeStruct((n, n), _F32),
        compiler_params=pltpu.CompilerParams(
            dimension_semantics=("arbitrary",)),
    )(src3, dst3)

    def blockdiag(al):
        return al.reshape(_NH * _HID, 1) * _KRON

    full = lambda shp: pl.BlockSpec(shp, lambda: tuple(0 for _ in shp))
    A, Bm = pl.pallas_call(
        _gat_body,
        in_specs=[full((n, x.shape[1])), full((x.shape[1], _NH * _HID)),
                  full((_NH * _HID, _NH)), full((_NH * _HID, _NH)),
                  full((1, _NH * _HID)),
                  full((_NH * _HID, _NH * _HID)),
                  full((_NH * _HID, _NH)), full((_NH * _HID, _NH)),
                  full((1, _NH * _HID)),
                  full((n, n)), full((_HID, _HID)), full((_HID, _HID)),
                  full((1, _HID))],
        out_specs=[full((n, _HID)), full((n, _HID))],
        out_shape=(jax.ShapeDtypeStruct((n, _HID), _F32),
                   jax.ShapeDtypeStruct((n, _HID), _F32)),
    )(x, W1, blockdiag(al1), blockdiag(ar1), b1.reshape(1, _NH * _HID),
      W2, blockdiag(al2), blockdiag(ar2), b2.reshape(1, _NH * _HID),
      C, l1W[:_HID], l1W[_HID:], l1b.reshape(1, _HID))

    bi = 32
    P = pl.pallas_call(
        _pair_body,
        grid=(n // bi,),
        in_specs=[pl.BlockSpec((bi, _HID), lambda i: (i, 0)),
                  pl.BlockSpec((n, _HID), lambda i: (0, 0)),
                  pl.BlockSpec((_HID, _HID), lambda i: (0, 0)),
                  pl.BlockSpec((1, _HID), lambda i: (0, 0)),
                  pl.BlockSpec((1, _HID), lambda i: (0, 0)),
                  pl.BlockSpec((1, 1), lambda i: (0, 0))],
        out_specs=pl.BlockSpec((bi * n, 1), lambda i: (i, 0)),
        out_shape=jax.ShapeDtypeStruct((n * n, 1), _F32),
    )(A, Bm, l2W, l2b.reshape(1, _HID), l3W.reshape(1, _HID),
      l3b.reshape(1, 1))
    return P.reshape(n * n)


# speedup vs baseline: 47.7465x; 1.8141x over previous
import numpy as np

import jax
import jax.numpy as jnp
from jax.experimental import pallas as pl
from jax.experimental.pallas import tpu as pltpu

_N = 512
_E = 16384
_NH = 4
_HID = 128
_F32 = jnp.float32


def _count_body(src_ref, dst_ref, c_ref):
    ec = src_ref.shape[-1]
    n = c_ref.shape[0]
    src = src_ref[...].reshape(1, ec)
    dst = dst_ref[...].reshape(1, ec)
    iota = jax.lax.broadcasted_iota(jnp.int32, (n, ec), 0)
    src_oh = (iota == src).astype(jnp.bfloat16)
    dst_oh = (iota == dst).astype(jnp.bfloat16)
    part = jax.lax.dot_general(dst_oh, src_oh, (((1,), (1,)), ((), ())),
                               preferred_element_type=_F32)

    @pl.when(pl.program_id(0) == 0)
    def _():
        c_ref[...] = part

    @pl.when(pl.program_id(0) != 0)
    def _():
        c_ref[...] += part


def _gat_body(x_ref, w1_ref, albd1_ref, arbd1_ref, b1_ref,
              w2_ref, albd2_ref, arbd2_ref, b2_ref,
              c_ref, l1wa_ref, l1wb_ref, l1b_ref,
              a_out_ref, b_out_ref):
    n = x_ref.shape[0]
    cm = c_ref[...]
    mask = cm > 0.0

    def attention(h, albd, arbd):
        el = jnp.dot(h, albd, preferred_element_type=_F32)
        er = jnp.dot(h, arbd, preferred_element_type=_F32)
        elt = el.T
        outs = []
        for hd in range(_NH):
            hh = h[:, hd * _HID:(hd + 1) * _HID]
            e = elt[hd:hd + 1, :] + er[:, hd:hd + 1]
            e = jnp.where(e >= 0.0, e, 0.2 * e)
            emax = jnp.max(jnp.where(mask, e, -1e30), axis=1, keepdims=True)
            emax = jnp.where(emax > -1e29, emax, 0.0)
            p = jnp.where(mask, jnp.exp(e - emax), 0.0) * cm
            denom = jnp.sum(p, axis=1, keepdims=True)
            denom = jnp.where(denom > 0.0, denom, 1.0)
            alpha = p / denom
            outs.append(jnp.dot(alpha, hh, preferred_element_type=_F32))
        return outs

    h1 = jnp.dot(x_ref[...], w1_ref[...], preferred_element_type=_F32)
    o1 = attention(h1, albd1_ref[...], arbd1_ref[...])
    acts = []
    for hd in range(_NH):
        v = o1[hd] + b1_ref[...][:, hd * _HID:(hd + 1) * _HID]
        acts.append(jnp.where(v > 0.0, v, jnp.exp(jnp.minimum(v, 0.0)) - 1.0))
    h2in = jnp.concatenate(acts, axis=1)

    h2 = jnp.dot(h2in, w2_ref[...], preferred_element_type=_F32)
    o2 = attention(h2, albd2_ref[...], arbd2_ref[...])
    hm = jnp.zeros((n, _HID), _F32)
    for hd in range(_NH):
        hm = hm + (o2[hd] + h2in[:, hd * _HID:(hd + 1) * _HID]
                   + b2_ref[...][:, hd * _HID:(hd + 1) * _HID])
    hm = hm * (1.0 / _NH)

    a_out_ref[...] = (jnp.dot(hm, l1wa_ref[...], preferred_element_type=_F32)
                      + l1b_ref[...])
    b_out_ref[...] = jnp.dot(hm, l1wb_ref[...], preferred_element_type=_F32)


def _pair_body(a_ref, b_ref, w2_ref, b2_ref, w3_ref, b3_ref, out_ref):
    bi = a_ref.shape[0]
    n = b_ref.shape[0]
    z = jnp.maximum(a_ref[...][:, None, :] + b_ref[...][None, :, :], 0.0)
    z = z.reshape(bi * n, _HID)
    q = jnp.dot(z, w2_ref[...], preferred_element_type=_F32) + b2_ref[...]
    q = jnp.maximum(q, 0.0)
    s = jax.lax.dot_general(w3_ref[...], q, (((1,), (1,)), ((), ())),
                            preferred_element_type=_F32) + b3_ref[...]
    sig = 1.0 / (1.0 + jnp.exp(-s))
    out_ref[...] = sig[0:1, :].reshape(1, 1, bi * n)


_KRON = np.kron(np.eye(_NH, dtype=np.float32), np.ones((_HID, 1), np.float32))


def kernel(x, edge_index, W1, al1, ar1, b1, W2, al2, ar2, b2,
           l1W, l1b, l2W, l2b, l3W, l3b):
    n, e = _N, _E

    nch = 8
    src3 = edge_index[0].reshape(nch, 1, e // nch)
    dst3 = edge_index[1].reshape(nch, 1, e // nch)
    C = pl.pallas_call(
        _count_body,
        grid=(nch,),
        in_specs=[pl.BlockSpec((1, 1, e // nch), lambda i: (i, 0, 0)),
                  pl.BlockSpec((1, 1, e // nch), lambda i: (i, 0, 0))],
        out_specs=pl.BlockSpec((n, n), lambda i: (0, 0)),
        out_shape=jax.ShapeDtypeStruct((n, n), _F32),
        compiler_params=pltpu.CompilerParams(
            dimension_semantics=("arbitrary",)),
    )(src3, dst3)

    def blockdiag(al):
        return al.reshape(_NH * _HID, 1) * _KRON

    full = lambda shp: pl.BlockSpec(shp, lambda: tuple(0 for _ in shp))
    A, Bm = pl.pallas_call(
        _gat_body,
        in_specs=[full((n, x.shape[1])), full((x.shape[1], _NH * _HID)),
                  full((_NH * _HID, _NH)), full((_NH * _HID, _NH)),
                  full((1, _NH * _HID)),
                  full((_NH * _HID, _NH * _HID)),
                  full((_NH * _HID, _NH)), full((_NH * _HID, _NH)),
                  full((1, _NH * _HID)),
                  full((n, n)), full((_HID, _HID)), full((_HID, _HID)),
                  full((1, _HID))],
        out_specs=[full((n, _HID)), full((n, _HID))],
        out_shape=(jax.ShapeDtypeStruct((n, _HID), _F32),
                   jax.ShapeDtypeStruct((n, _HID), _F32)),
    )(x, W1, blockdiag(al1), blockdiag(ar1), b1.reshape(1, _NH * _HID),
      W2, blockdiag(al2), blockdiag(ar2), b2.reshape(1, _NH * _HID),
      C, l1W[:_HID], l1W[_HID:], l1b.reshape(1, _HID))

    bi = 32
    w3rep = jnp.broadcast_to(l3W.reshape(1, _HID), (8, _HID))
    P = pl.pallas_call(
        _pair_body,
        grid=(n // bi,),
        in_specs=[pl.BlockSpec((bi, _HID), lambda i: (i, 0)),
                  pl.BlockSpec((n, _HID), lambda i: (0, 0)),
                  pl.BlockSpec((_HID, _HID), lambda i: (0, 0)),
                  pl.BlockSpec((1, _HID), lambda i: (0, 0)),
                  pl.BlockSpec((8, _HID), lambda i: (0, 0)),
                  pl.BlockSpec((1, 1), lambda i: (0, 0))],
        out_specs=pl.BlockSpec((1, 1, bi * n), lambda i: (i, 0, 0)),
        out_shape=jax.ShapeDtypeStruct((n // bi, 1, bi * n), _F32),
    )(A, Bm, l2W, l2b.reshape(1, _HID), w3rep, l3b.reshape(1, 1))
    return P.reshape(n * n)
